# baseline (speedup 1.0000x reference)
import functools

import numpy as np
import jax
import jax.numpy as jnp
from jax import lax
from jax.experimental import pallas as pl
from jax.experimental.pallas import tpu as pltpu
from jax.experimental.pallas import tpu_sc as plsc

_VOCAB = 100000
_SEQ = 2048
_D = 512
_B = 4
_NC = 2
_NS = 16
_NW = _NC * _NS
_PW = _SEQ // _NW
_VPR = _D // 16
_HR = _PW // 2


def _pos_encoding():
    denom = np.power(10000.0, 2.0 * np.arange(_D // 2, dtype=np.float64) / _D)
    inv = jnp.asarray(1.0 / denom, dtype=jnp.float32)
    pos = jax.lax.iota(jnp.float32, _SEQ)
    ang = pos[:, None] * inv[None, :]
    pe = jnp.stack([jnp.sin(ang), jnp.cos(ang)], axis=-1)
    return pe.reshape(_SEQ, _D)


_mesh = plsc.VectorSubcoreMesh(core_axis_name="c", subcore_axis_name="s")


@functools.partial(
    pl.kernel,
    mesh=_mesh,
    out_type=jax.ShapeDtypeStruct((_B * _SEQ, _D), jnp.float32),
    scratch_types=[
        pltpu.VMEM((_B, _PW), jnp.int32),
        pltpu.VMEM((_PW, _D), jnp.float32),
        pltpu.VMEM((_PW, _D), jnp.float32),
        pltpu.VMEM((_PW, _D), jnp.float32),
        pltpu.SemaphoreType.DMA,
        pltpu.SemaphoreType.DMA,
        pltpu.SemaphoreType.DMA,
        pltpu.SemaphoreType.DMA,
        pltpu.SemaphoreType.DMA,
    ],
)
def _emb_kernel(idx_hbm, table_hbm, pos_hbm, out_hbm,
                idx_v, pos_v, rv0, rv1, gs0, gs1, os0, os1, ps):
    c = lax.axis_index("c")
    s = lax.axis_index("s")
    w = s * _NC + c
    p0 = w * _PW

    rv = (rv0, rv1)
    gs = (gs0, gs1)
    osem = (os0, os1)

    pd = pltpu.async_copy(pos_hbm.at[pl.ds(p0, _PW)], pos_v, ps)

    def gather(b):
        return pltpu.async_copy(table_hbm.at[idx_v.at[b]], rv[b % 2], gs[b % 2])

    def out_half(b, h):
        row0 = b * _SEQ + p0 + h * _HR
        return pltpu.async_copy(
            rv[b % 2].at[pl.ds(h * _HR, _HR)],
            out_hbm.at[pl.ds(row0, _HR)], osem[b % 2])

    def add_half(b, h):
        row_ref = rv[b % 2]

        def body(r):
            for j in range(_VPR):
                v = pos_v[r, pl.ds(j * 16, 16)]
                plsc.addupdate(row_ref.at[r, pl.ds(j * 16, 16)], v)

        plsc.parallel_loop(h * _HR, (h + 1) * _HR, unroll=2)(body)

    gd = [None] * _B
    od = [[None, None] for _ in range(_B)]
    pltpu.sync_copy(idx_hbm.at[0, pl.ds(p0, _PW)], idx_v.at[0])
    gd[0] = gather(0)
    pltpu.sync_copy(idx_hbm.at[1, pl.ds(p0, _PW)], idx_v.at[1])
    gd[1] = gather(1)
    pltpu.sync_copy(idx_hbm.at[2, pl.ds(p0, _PW)], idx_v.at[2])
    pltpu.sync_copy(idx_hbm.at[3, pl.ds(p0, _PW)], idx_v.at[3])
    pd.wait()
    for b in range(_B):
        gd[b].wait()
        add_half(b, 0)
        od[b][0] = out_half(b, 0)
        add_half(b, 1)
        od[b][1] = out_half(b, 1)
        if b + 2 < _B:
            od[b][0].wait()
            od[b][1].wait()
            gd[b + 2] = gather(b + 2)
    od[_B - 2][0].wait()
    od[_B - 2][1].wait()
    od[_B - 1][0].wait()
    od[_B - 1][1].wait()


def kernel(inputs, table):
    out = _emb_kernel(inputs.astype(jnp.int32), table, _pos_encoding())
    return out.reshape(_B, _SEQ, _D)


# speedup vs baseline: 1.3536x; 1.3536x over previous
import functools

import numpy as np
import jax
import jax.numpy as jnp
from jax import lax
from jax.experimental import pallas as pl
from jax.experimental.pallas import tpu as pltpu
from jax.experimental.pallas import tpu_sc as plsc

_VOCAB = 100000
_SEQ = 2048
_D = 512
_B = 4
_NC = 2
_NS = 16
_NW = _NC * _NS
_PW = _SEQ // _NW
_VPR = _D // 16
_HR = _PW // 2


def _pos_encoding():
    i = np.arange(_D // 2, dtype=np.float64)
    denom = np.power(10000.0, 2.0 * i / _D)
    pos = np.arange(_SEQ, dtype=np.float64)[:, None]
    pe = np.zeros((_SEQ, _D), dtype=np.float64)
    pe[:, 0::2] = np.sin(pos / denom)
    pe[:, 1::2] = np.cos(pos / denom)
    return jnp.asarray(pe, dtype=jnp.float32)


_mesh = plsc.VectorSubcoreMesh(core_axis_name="c", subcore_axis_name="s")


@functools.partial(
    pl.kernel,
    mesh=_mesh,
    out_type=jax.ShapeDtypeStruct((_B * _SEQ, _D), jnp.float32),
    scratch_types=[
        pltpu.VMEM((_B, _PW), jnp.int32),
        pltpu.VMEM((_PW, _D), jnp.float32),
        pltpu.VMEM((_PW, _D), jnp.float32),
        pltpu.VMEM((_PW, _D), jnp.float32),
        pltpu.SemaphoreType.DMA,
        pltpu.SemaphoreType.DMA,
        pltpu.SemaphoreType.DMA,
        pltpu.SemaphoreType.DMA,
        pltpu.SemaphoreType.DMA,
    ],
)
def _emb_kernel(idx_hbm, table_hbm, pos_hbm, out_hbm,
                idx_v, pos_v, rv0, rv1, gs0, gs1, os0, os1, ps):
    c = lax.axis_index("c")
    s = lax.axis_index("s")
    w = s * _NC + c
    p0 = w * _PW

    rv = (rv0, rv1)
    gs = (gs0, gs1)
    osem = (os0, os1)

    def gather(b, i):
        return pltpu.async_copy(table_hbm.at[idx_v.at[b]], rv[i], gs[i])

    def gather_wait(i):
        pltpu.make_async_copy(table_hbm.at[idx_v.at[0]], rv[i], gs[i]).wait()

    def out_half(b, i, h):
        row0 = b * _SEQ + p0 + h * _HR
        return pltpu.async_copy(
            rv[i].at[pl.ds(h * _HR, _HR)],
            out_hbm.at[pl.ds(row0, _HR)], osem[i])

    def out_wait(i):
        pltpu.make_async_copy(
            rv[i].at[pl.ds(0, _HR)], out_hbm.at[pl.ds(0, _HR)], osem[i]).wait()

    def add_half(i, h):
        row_ref = rv[i]

        def body(r):
            for j in range(_VPR):
                v = pos_v[r, pl.ds(j * 16, 16)]
                plsc.addupdate(row_ref.at[r, pl.ds(j * 16, 16)], v)

        plsc.parallel_loop(h * _HR, (h + 1) * _HR, unroll=2)(body)

    pd = pltpu.async_copy(pos_hbm.at[pl.ds(p0, _PW)], pos_v, ps)
    pltpu.sync_copy(idx_hbm.at[0, pl.ds(p0, _PW)], idx_v.at[0])
    gather(0, 0)
    pltpu.sync_copy(idx_hbm.at[1, pl.ds(p0, _PW)], idx_v.at[1])
    gather(1, 1)
    pltpu.sync_copy(idx_hbm.at[2, pl.ds(p0, _PW)], idx_v.at[2])
    pltpu.sync_copy(idx_hbm.at[3, pl.ds(p0, _PW)], idx_v.at[3])
    pd.wait()

    @pl.loop(0, _B, step=2)
    def _pair(b):
        for i in range(2):
            gather_wait(i)
            add_half(i, 0)
            out_half(b + i, i, 0)
            add_half(i, 1)
            out_half(b + i, i, 1)

        @pl.when(b + 2 < _B)
        def _():
            for i in range(2):
                out_wait(i)
                out_wait(i)
                gather(b + 2 + i, i)

    for i in range(2):
        out_wait(i)
        out_wait(i)


def kernel(inputs, table):
    out = _emb_kernel(inputs.astype(jnp.int32), table, _pos_encoding())
    return out.reshape(_B, _SEQ, _D)


# speedup vs baseline: 1.4658x; 1.0829x over previous
import functools

import numpy as np
import jax
import jax.numpy as jnp
from jax import lax
from jax.experimental import pallas as pl
from jax.experimental.pallas import tpu as pltpu
from jax.experimental.pallas import tpu_sc as plsc

_VOCAB = 100000
_SEQ = 2048
_D = 512
_B = 4
_NC = 2
_NS = 16
_NW = _NC * _NS
_PW = _SEQ // _NW
_VPR = _D // 16
_HR = _PW // 2


def _pos_encoding():
    i = np.arange(_D // 2, dtype=np.float64)
    denom = np.power(10000.0, 2.0 * i / _D)
    pos = np.arange(_SEQ, dtype=np.float64)[:, None]
    pe = np.zeros((_SEQ, _D), dtype=np.float64)
    pe[:, 0::2] = np.sin(pos / denom)
    pe[:, 1::2] = np.cos(pos / denom)
    return jnp.asarray(pe, dtype=jnp.float32)


_mesh = plsc.VectorSubcoreMesh(core_axis_name="c", subcore_axis_name="s")


@functools.partial(
    pl.kernel,
    mesh=_mesh,
    out_type=jax.ShapeDtypeStruct((_B * _SEQ, _D), jnp.float32),
    scratch_types=[
        pltpu.VMEM((_B, _PW), jnp.int32),
        pltpu.VMEM((_PW, _D), jnp.float32),
        pltpu.VMEM((_PW, _D), jnp.float32),
        pltpu.VMEM((_PW, _D), jnp.float32),
        pltpu.SemaphoreType.DMA,
        pltpu.SemaphoreType.DMA,
        pltpu.SemaphoreType.DMA,
        pltpu.SemaphoreType.DMA,
        pltpu.SemaphoreType.DMA,
    ],
)
def _emb_kernel(idx_hbm, table_hbm, pos_hbm, out_hbm,
                idx_v, pos_v, rv0, rv1, gs0, gs1, os0, os1, ps):
    c = lax.axis_index("c")
    s = lax.axis_index("s")
    w = s * _NC + c
    p0 = w * _PW

    rv = (rv0, rv1)
    gs = (gs0, gs1)
    osem = (os0, os1)

    def gather(b, i):
        return pltpu.async_copy(table_hbm.at[idx_v.at[b]], rv[i], gs[i])

    def gather_wait(i):
        pltpu.make_async_copy(table_hbm.at[idx_v.at[0]], rv[i], gs[i]).wait()

    def out_half(b, i, h):
        row0 = b * _SEQ + p0 + h * _HR
        return pltpu.async_copy(
            rv[i].at[pl.ds(h * _HR, _HR)],
            out_hbm.at[pl.ds(row0, _HR)], osem[i])

    def out_wait(i):
        pltpu.make_async_copy(
            rv[i].at[pl.ds(0, _HR)], out_hbm.at[pl.ds(0, _HR)], osem[i]).wait()

    def add_half(i, h):
        row_ref = rv[i]

        def body(r):
            for j in range(_VPR):
                v = pos_v[r, pl.ds(j * 16, 16)]
                plsc.addupdate(row_ref.at[r, pl.ds(j * 16, 16)], v)

        plsc.parallel_loop(h * _HR, (h + 1) * _HR)(body)

    pd = pltpu.async_copy(pos_hbm.at[pl.ds(p0, _PW)], pos_v, ps)
    pltpu.sync_copy(idx_hbm.at[0, pl.ds(p0, _PW)], idx_v.at[0])
    gather(0, 0)
    pltpu.sync_copy(idx_hbm.at[1, pl.ds(p0, _PW)], idx_v.at[1])
    gather(1, 1)
    pltpu.sync_copy(idx_hbm.at[2, pl.ds(p0, _PW)], idx_v.at[2])
    pltpu.sync_copy(idx_hbm.at[3, pl.ds(p0, _PW)], idx_v.at[3])
    pd.wait()

    @pl.loop(0, _B, step=2)
    def _pair(b):
        for i in range(2):
            gather_wait(i)
            add_half(i, 0)
            out_half(b + i, i, 0)
            add_half(i, 1)
            out_half(b + i, i, 1)

        @pl.when(b + 2 < _B)
        def _():
            for i in range(2):
                out_wait(i)
                out_wait(i)
                gather(b + 2 + i, i)

    for i in range(2):
        out_wait(i)
        out_wait(i)


def kernel(inputs, table):
    out = _emb_kernel(inputs.astype(jnp.int32), table, _pos_encoding())
    return out.reshape(_B, _SEQ, _D)
